# Initial kernel scaffold; baseline (speedup 1.0000x reference)
import jax
import jax.numpy as jnp
from jax.experimental import pallas as pl


def kernel(P2, P3, P4, P5, rois, fc1_w, fc1_b, fc2_w, fc2_b, cls_w, cls_b, reg_w, reg_b):
    raise NotImplementedError("write your pallas kernel here")


# speedup vs baseline: 5.8264x; 5.8264x over previous
import jax
import jax.numpy as jnp
from jax.experimental import pallas as pl
from jax.experimental.pallas import tpu as pltpu

NP = 1024
RPAD = 87048
KB = 1792
KSTEPS = 7
NB = 256
IMG = 1024.0


def _prep(rois):
    y1, x1, y2, x2 = rois[:, 0], rois[:, 1], rois[:, 2], rois[:, 3]
    h = y2 - y1
    w = x2 - x1
    lvl = jnp.floor(4.0 + jnp.log2(jnp.sqrt(jnp.maximum(h * w, 1e-6)) / 224.0))
    li = jnp.clip(lvl, 2.0, 5.0).astype(jnp.int32) - 2
    stride = jnp.array([4.0, 8.0, 16.0, 32.0], jnp.float32)[li]
    hw = jnp.array([256, 128, 64, 32], jnp.int32)[li]
    off = jnp.array([0, 65536, 81920, 86016], jnp.int32)[li]
    s = rois / stride[:, None]
    grid = (jnp.arange(7, dtype=jnp.float32) + 0.5) / 7.0
    ys = s[:, 0:1] + grid[None, :] * (s[:, 2:3] - s[:, 0:1])
    xs = s[:, 1:2] + grid[None, :] * (s[:, 3:4] - s[:, 1:2])
    hwf = hw.astype(jnp.float32)[:, None]
    y0f = jnp.clip(jnp.floor(ys), 0.0, hwf - 1.0)
    x0f = jnp.clip(jnp.floor(xs), 0.0, hwf - 1.0)
    y0 = y0f.astype(jnp.int32)
    x0 = x0f.astype(jnp.int32)
    y1i = jnp.minimum(y0 + 1, hw[:, None] - 1)
    x1i = jnp.minimum(x0 + 1, hw[:, None] - 1)
    wy = jnp.clip(ys - y0f, 0.0, 1.0)
    wx = jnp.clip(xs - x0f, 0.0, 1.0)
    yb0 = off[:, None] + y0 * hw[:, None]
    yb1 = off[:, None] + y1i * hw[:, None]
    idx = jnp.concatenate([yb0, yb1, x0], axis=1)
    xe = x1i == x0
    cl = jnp.where(xe, 1.0, 1.0 - wx)
    ch = jnp.where(xe, 0.0, wx)
    a = (1.0 - wy)[:, :, None]
    b = wy[:, :, None]
    w0e = jnp.stack([a * cl[:, None, :], a * ch[:, None, :]], axis=-1)
    w1e = jnp.stack([b * cl[:, None, :], b * ch[:, None, :]], axis=-1)
    w0e = w0e.reshape(NP, 98, 1)
    w1e = w1e.reshape(NP, 98, 1)
    return idx, w0e, w1e


def _align_body(idx_sm, f_ref, w0_ref, w1_ref, out_ref, c0, c1):
    i = pl.program_id(0)
    xk = [idx_sm[i, 14 + k] for k in range(7)]
    for j in range(7):
        r0 = idx_sm[i, j]
        r1 = idx_sm[i, 7 + j]
        for k in range(7):
            p = j * 7 + k
            c0[2 * p:2 * p + 2, :] = f_ref[pl.ds(r0 + xk[k], 2), 0, :]
            c1[2 * p:2 * p + 2, :] = f_ref[pl.ds(r1 + xk[k], 2), 0, :]
    t = c0[:] * w0_ref[0] + c1[:] * w1_ref[0]
    out_ref[0] = t.reshape(49, 2, 128).sum(axis=1).astype(out_ref.dtype)


def _align(f_half, pre, out_dtype):
    idx, w0e, w1e = pre
    grid_spec = pltpu.PrefetchScalarGridSpec(
        num_scalar_prefetch=1,
        grid=(NP,),
        in_specs=[
            pl.BlockSpec(memory_space=pltpu.MemorySpace.VMEM),
            pl.BlockSpec((1, 98, 1), lambda i, *_: (i, 0, 0)),
            pl.BlockSpec((1, 98, 1), lambda i, *_: (i, 0, 0)),
        ],
        out_specs=pl.BlockSpec((1, 49, 128), lambda i, *_: (i, 0, 0)),
        scratch_shapes=[pltpu.VMEM((98, 128), jnp.float32)] * 2,
    )
    return pl.pallas_call(
        _align_body,
        grid_spec=grid_spec,
        out_shape=jax.ShapeDtypeStruct((NP, 49, 128), out_dtype),
        compiler_params=pltpu.CompilerParams(
            dimension_semantics=("parallel",),
            vmem_limit_bytes=60 * 1024 * 1024,
        ),
    )(idx, f_half, w0e, w1e)


def _head_delta_body(x_ref, w1_ref, b1_ref, w2_ref, b2_ref, wr_ref, br_ref,
                     rois_ref, out_ref, acc):
    k = pl.program_id(1)

    @pl.when(k == 0)
    def _():
        acc[...] = jnp.zeros_like(acc)

    acc[...] += jnp.dot(x_ref[...].astype(jnp.bfloat16),
                        w1_ref[...].astype(jnp.bfloat16),
                        preferred_element_type=jnp.float32)

    @pl.when(k == KSTEPS - 1)
    def _():
        h1 = jnp.maximum(acc[...] + b1_ref[0], 0.0)
        h2 = jnp.maximum(
            jnp.dot(h1.astype(jnp.bfloat16), w2_ref[...].astype(jnp.bfloat16),
                    preferred_element_type=jnp.float32) + b2_ref[0], 0.0)
        dts = jnp.dot(h2.astype(jnp.bfloat16), wr_ref[...].astype(jnp.bfloat16),
                      preferred_element_type=jnp.float32) + br_ref[0]
        r = rois_ref[...]
        yy1, xx1, yy2, xx2 = r[:, 0:1], r[:, 1:2], r[:, 2:3], r[:, 3:4]
        hh = yy2 - yy1
        ww = xx2 - xx1
        cy = yy1 + 0.5 * hh + (dts[:, 0:1] * 0.1) * hh
        cx = xx1 + 0.5 * ww + (dts[:, 1:2] * 0.1) * ww
        nh = hh * jnp.exp(dts[:, 2:3] * 0.2)
        nw = ww * jnp.exp(dts[:, 3:4] * 0.2)
        nb = jnp.concatenate(
            [cy - 0.5 * nh, cx - 0.5 * nw, cy + 0.5 * nh, cx + 0.5 * nw], axis=1)
        out_ref[...] = jnp.clip(nb, 0.0, IMG)


def _head_delta(x, w1, b1, w2, b2, wr, br, rois):
    return pl.pallas_call(
        _head_delta_body,
        grid=(NP // NB, KSTEPS),
        in_specs=[
            pl.BlockSpec((NB, KB), lambda n, k: (n, k)),
            pl.BlockSpec((KB, 1024), lambda n, k: (k, 0)),
            pl.BlockSpec((1, 1024), lambda n, k: (0, 0)),
            pl.BlockSpec((1024, 1024), lambda n, k: (0, 0)),
            pl.BlockSpec((1, 1024), lambda n, k: (0, 0)),
            pl.BlockSpec((1024, 128), lambda n, k: (0, 0)),
            pl.BlockSpec((1, 128), lambda n, k: (0, 0)),
            pl.BlockSpec((NB, 4), lambda n, k: (n, 0)),
        ],
        out_specs=pl.BlockSpec((NB, 4), lambda n, k: (n, 0)),
        out_shape=jax.ShapeDtypeStruct((NP, 4), jnp.float32),
        scratch_shapes=[pltpu.VMEM((NB, 1024), jnp.float32)],
        compiler_params=pltpu.CompilerParams(
            dimension_semantics=("parallel", "arbitrary"),
            vmem_limit_bytes=56 * 1024 * 1024,
        ),
    )(x, w1, b1, w2, b2, wr, br, rois)


def _head_ens_body(x_ref, w1_ref, b1_ref, w2_ref, b2_ref, wc_ref, bc_ref,
                   out_ref, acc):
    h = pl.program_id(1)
    k = pl.program_id(2)

    @pl.when(k == 0)
    def _():
        acc[...] = jnp.zeros_like(acc)

    acc[...] += jnp.dot(x_ref[...],
                        w1_ref[0].astype(jnp.bfloat16),
                        preferred_element_type=jnp.float32)

    @pl.when(k == KSTEPS - 1)
    def _():
        h1 = jnp.maximum(acc[...] + b1_ref[0, 0], 0.0)
        h2 = jnp.maximum(
            jnp.dot(h1.astype(jnp.bfloat16), w2_ref[0].astype(jnp.bfloat16),
                    preferred_element_type=jnp.float32) + b2_ref[0, 0], 0.0)
        lg = jnp.dot(h2.astype(jnp.bfloat16), wc_ref[0].astype(jnp.bfloat16),
                     preferred_element_type=jnp.float32) + bc_ref[0, 0]
        m = jnp.max(lg, axis=-1, keepdims=True)
        e = jnp.exp(lg - m)
        p = e / (3.0 * jnp.sum(e, axis=-1, keepdims=True))

        @pl.when(h == 0)
        def _():
            out_ref[...] = p

        @pl.when(h != 0)
        def _():
            out_ref[...] = out_ref[...] + p


def _head_ens(x, w1, b1, w2, b2, wc, bc):
    return pl.pallas_call(
        _head_ens_body,
        grid=(NP // NB, 3, KSTEPS),
        in_specs=[
            pl.BlockSpec((NB, KB), lambda n, h, k: (n, k)),
            pl.BlockSpec((1, KB, 1024), lambda n, h, k: (h, k, 0)),
            pl.BlockSpec((1, 1, 1024), lambda n, h, k: (h, 0, 0)),
            pl.BlockSpec((1, 1024, 1024), lambda n, h, k: (h, 0, 0)),
            pl.BlockSpec((1, 1, 1024), lambda n, h, k: (h, 0, 0)),
            pl.BlockSpec((1, 1024, 128), lambda n, h, k: (h, 0, 0)),
            pl.BlockSpec((1, 1, 128), lambda n, h, k: (h, 0, 0)),
        ],
        out_specs=pl.BlockSpec((NB, 128), lambda n, h, k: (n, 0)),
        out_shape=jax.ShapeDtypeStruct((NP, 128), jnp.float32),
        scratch_shapes=[pltpu.VMEM((NB, 1024), jnp.float32)],
        compiler_params=pltpu.CompilerParams(
            dimension_semantics=("parallel", "arbitrary", "arbitrary"),
            vmem_limit_bytes=56 * 1024 * 1024,
        ),
    )(x, w1, b1, w2, b2, wc, bc)


def kernel(P2, P3, P4, P5, rois, fc1_w, fc1_b, fc2_w, fc2_b, cls_w, cls_b,
           reg_w, reg_b):
    f = jnp.concatenate([
        P2[0].reshape(65536, 256), P3[0].reshape(16384, 256),
        P4[0].reshape(4096, 256), P5[0].reshape(1024, 256)], axis=0)
    f = jnp.pad(f, ((0, RPAD - 87040), (0, 0)))
    fa = f[:, :128].reshape(RPAD, 1, 128)
    fb = f[:, 128:].reshape(RPAD, 1, 128)

    r = jnp.pad(rois, ((0, NP - rois.shape[0]), (0, 0)))

    for i in range(2):
        pre = _prep(r)
        plo = _align(fa, pre, jnp.float32)
        phi = _align(fb, pre, jnp.float32)
        x = jnp.stack([plo, phi], axis=2).reshape(NP, 12544)
        wr = jnp.pad(reg_w[i], ((0, 0), (0, 124)))
        br = jnp.pad(reg_b[i], (0, 124)).reshape(1, 128)
        r = _head_delta(x, fc1_w[i], fc1_b[i].reshape(1, 1024), fc2_w[i],
                        fc2_b[i].reshape(1, 1024), wr, br, r)

    pre = _prep(r)
    plo = _align(fa, pre, jnp.bfloat16)
    phi = _align(fb, pre, jnp.bfloat16)
    x = jnp.stack([plo, phi], axis=2).reshape(NP, 12544)
    wc = jnp.pad(cls_w, ((0, 0), (0, 0), (0, 47)))
    bc = jnp.where(
        jnp.arange(128)[None, :] < 81,
        jnp.pad(cls_b, ((0, 0), (0, 47))), -1e9).reshape(3, 1, 128)
    probs = _head_ens(x, fc1_w, fc1_b.reshape(3, 1, 1024), fc2_w,
                      fc2_b.reshape(3, 1, 1024), wc, bc)
    return probs[:1000, :81]


# speedup vs baseline: 11.5233x; 1.9778x over previous
import jax
import jax.numpy as jnp
from jax.experimental import pallas as pl
from jax.experimental.pallas import tpu as pltpu

NP = 1024
RPAD = 87048
KB = 1792
KSTEPS = 7
NB = 256
IMG = 1024.0


def _prep(rois):
    y1, x1, y2, x2 = rois[:, 0], rois[:, 1], rois[:, 2], rois[:, 3]
    h = y2 - y1
    w = x2 - x1
    lvl = jnp.floor(4.0 + jnp.log2(jnp.sqrt(jnp.maximum(h * w, 1e-6)) / 224.0))
    li = jnp.clip(lvl, 2.0, 5.0).astype(jnp.int32) - 2
    stride = jnp.array([4.0, 8.0, 16.0, 32.0], jnp.float32)[li]
    hw = jnp.array([256, 128, 64, 32], jnp.int32)[li]
    off = jnp.array([0, 65536, 81920, 86016], jnp.int32)[li]
    s = rois / stride[:, None]
    grid = (jnp.arange(7, dtype=jnp.float32) + 0.5) / 7.0
    ys = s[:, 0:1] + grid[None, :] * (s[:, 2:3] - s[:, 0:1])
    xs = s[:, 1:2] + grid[None, :] * (s[:, 3:4] - s[:, 1:2])
    hwf = hw.astype(jnp.float32)[:, None]
    y0f = jnp.clip(jnp.floor(ys), 0.0, hwf - 1.0)
    x0f = jnp.clip(jnp.floor(xs), 0.0, hwf - 1.0)
    y0 = y0f.astype(jnp.int32)
    x0 = x0f.astype(jnp.int32)
    y1i = jnp.minimum(y0 + 1, hw[:, None] - 1)
    x1i = jnp.minimum(x0 + 1, hw[:, None] - 1)
    wy = jnp.clip(ys - y0f, 0.0, 1.0)
    wx = jnp.clip(xs - x0f, 0.0, 1.0)
    yb0 = off[:, None] + y0 * hw[:, None]
    yb1 = off[:, None] + y1i * hw[:, None]
    idx = jnp.concatenate([yb0, yb1, x0], axis=1)
    xe = x1i == x0
    cl = jnp.where(xe, 1.0, 1.0 - wx)
    ch = jnp.where(xe, 0.0, wx)
    a = (1.0 - wy)[:, :, None]
    b = wy[:, :, None]
    w0e = jnp.stack([a * cl[:, None, :], a * ch[:, None, :]], axis=-1)
    w1e = jnp.stack([b * cl[:, None, :], b * ch[:, None, :]], axis=-1)
    w0e = w0e.reshape(NP, 98, 1)
    w1e = w1e.reshape(NP, 98, 1)
    return idx, w0e, w1e


BA = 8


def _align_body(idx_sm, f_ref, w0_ref, w1_ref, out_ref, c0, c1):
    i = pl.program_id(0)
    for b in range(BA):
        row = i * BA + b
        xk = [idx_sm[row, 14 + k] for k in range(7)]
        for j in range(7):
            r0 = idx_sm[row, j]
            r1 = idx_sm[row, 7 + j]
            for k in range(7):
                p = b * 98 + 2 * (j * 7 + k)
                c0[p:p + 2, :] = f_ref[pl.ds(r0 + xk[k], 2), 0, :]
                c1[p:p + 2, :] = f_ref[pl.ds(r1 + xk[k], 2), 0, :]
    for b in range(BA):
        t = (c0[b * 98:(b + 1) * 98, :] * w0_ref[b]
             + c1[b * 98:(b + 1) * 98, :] * w1_ref[b])
        out_ref[b] = t.reshape(49, 2, 128).sum(axis=1).astype(out_ref.dtype)


def _align(f_half, pre, out_dtype):
    idx, w0e, w1e = pre
    grid_spec = pltpu.PrefetchScalarGridSpec(
        num_scalar_prefetch=1,
        grid=(NP // BA,),
        in_specs=[
            pl.BlockSpec(memory_space=pltpu.MemorySpace.VMEM),
            pl.BlockSpec((BA, 98, 1), lambda i, *_: (i, 0, 0)),
            pl.BlockSpec((BA, 98, 1), lambda i, *_: (i, 0, 0)),
        ],
        out_specs=pl.BlockSpec((BA, 49, 128), lambda i, *_: (i, 0, 0)),
        scratch_shapes=[pltpu.VMEM((BA * 98, 128), jnp.float32)] * 2,
    )
    return pl.pallas_call(
        _align_body,
        grid_spec=grid_spec,
        out_shape=jax.ShapeDtypeStruct((NP, 49, 128), out_dtype),
        compiler_params=pltpu.CompilerParams(
            dimension_semantics=("parallel",),
            vmem_limit_bytes=60 * 1024 * 1024,
        ),
    )(idx, f_half, w0e, w1e)


def _head_delta_body(x_ref, w1_ref, b1_ref, w2_ref, b2_ref, wr_ref, br_ref,
                     rois_ref, out_ref, acc):
    k = pl.program_id(1)

    @pl.when(k == 0)
    def _():
        acc[...] = jnp.zeros_like(acc)

    acc[...] += jnp.dot(x_ref[...].astype(jnp.bfloat16),
                        w1_ref[...].astype(jnp.bfloat16),
                        preferred_element_type=jnp.float32)

    @pl.when(k == KSTEPS - 1)
    def _():
        h1 = jnp.maximum(acc[...] + b1_ref[0], 0.0)
        h2 = jnp.maximum(
            jnp.dot(h1.astype(jnp.bfloat16), w2_ref[...].astype(jnp.bfloat16),
                    preferred_element_type=jnp.float32) + b2_ref[0], 0.0)
        dts = jnp.dot(h2.astype(jnp.bfloat16), wr_ref[...].astype(jnp.bfloat16),
                      preferred_element_type=jnp.float32) + br_ref[0]
        r = rois_ref[...]
        yy1, xx1, yy2, xx2 = r[:, 0:1], r[:, 1:2], r[:, 2:3], r[:, 3:4]
        hh = yy2 - yy1
        ww = xx2 - xx1
        cy = yy1 + 0.5 * hh + (dts[:, 0:1] * 0.1) * hh
        cx = xx1 + 0.5 * ww + (dts[:, 1:2] * 0.1) * ww
        nh = hh * jnp.exp(dts[:, 2:3] * 0.2)
        nw = ww * jnp.exp(dts[:, 3:4] * 0.2)
        nb = jnp.concatenate(
            [cy - 0.5 * nh, cx - 0.5 * nw, cy + 0.5 * nh, cx + 0.5 * nw], axis=1)
        out_ref[...] = jnp.clip(nb, 0.0, IMG)


def _head_delta(x, w1, b1, w2, b2, wr, br, rois):
    return pl.pallas_call(
        _head_delta_body,
        grid=(NP // NB, KSTEPS),
        in_specs=[
            pl.BlockSpec((NB, KB), lambda n, k: (n, k)),
            pl.BlockSpec((KB, 1024), lambda n, k: (k, 0)),
            pl.BlockSpec((1, 1024), lambda n, k: (0, 0)),
            pl.BlockSpec((1024, 1024), lambda n, k: (0, 0)),
            pl.BlockSpec((1, 1024), lambda n, k: (0, 0)),
            pl.BlockSpec((1024, 128), lambda n, k: (0, 0)),
            pl.BlockSpec((1, 128), lambda n, k: (0, 0)),
            pl.BlockSpec((NB, 4), lambda n, k: (n, 0)),
        ],
        out_specs=pl.BlockSpec((NB, 4), lambda n, k: (n, 0)),
        out_shape=jax.ShapeDtypeStruct((NP, 4), jnp.float32),
        scratch_shapes=[pltpu.VMEM((NB, 1024), jnp.float32)],
        compiler_params=pltpu.CompilerParams(
            dimension_semantics=("parallel", "arbitrary"),
            vmem_limit_bytes=56 * 1024 * 1024,
        ),
    )(x, w1, b1, w2, b2, wr, br, rois)


def _head_ens_body(x_ref, w1_ref, b1_ref, w2_ref, b2_ref, wc_ref, bc_ref,
                   out_ref, acc):
    h = pl.program_id(1)
    k = pl.program_id(2)

    @pl.when(k == 0)
    def _():
        acc[...] = jnp.zeros_like(acc)

    acc[...] += jnp.dot(x_ref[...],
                        w1_ref[0].astype(jnp.bfloat16),
                        preferred_element_type=jnp.float32)

    @pl.when(k == KSTEPS - 1)
    def _():
        h1 = jnp.maximum(acc[...] + b1_ref[0, 0], 0.0)
        h2 = jnp.maximum(
            jnp.dot(h1.astype(jnp.bfloat16), w2_ref[0].astype(jnp.bfloat16),
                    preferred_element_type=jnp.float32) + b2_ref[0, 0], 0.0)
        lg = jnp.dot(h2.astype(jnp.bfloat16), wc_ref[0].astype(jnp.bfloat16),
                     preferred_element_type=jnp.float32) + bc_ref[0, 0]
        m = jnp.max(lg, axis=-1, keepdims=True)
        e = jnp.exp(lg - m)
        p = e / (3.0 * jnp.sum(e, axis=-1, keepdims=True))

        @pl.when(h == 0)
        def _():
            out_ref[...] = p

        @pl.when(h != 0)
        def _():
            out_ref[...] = out_ref[...] + p


def _head_ens(x, w1, b1, w2, b2, wc, bc):
    return pl.pallas_call(
        _head_ens_body,
        grid=(NP // NB, 3, KSTEPS),
        in_specs=[
            pl.BlockSpec((NB, KB), lambda n, h, k: (n, k)),
            pl.BlockSpec((1, KB, 1024), lambda n, h, k: (h, k, 0)),
            pl.BlockSpec((1, 1, 1024), lambda n, h, k: (h, 0, 0)),
            pl.BlockSpec((1, 1024, 1024), lambda n, h, k: (h, 0, 0)),
            pl.BlockSpec((1, 1, 1024), lambda n, h, k: (h, 0, 0)),
            pl.BlockSpec((1, 1024, 128), lambda n, h, k: (h, 0, 0)),
            pl.BlockSpec((1, 1, 128), lambda n, h, k: (h, 0, 0)),
        ],
        out_specs=pl.BlockSpec((NB, 128), lambda n, h, k: (n, 0)),
        out_shape=jax.ShapeDtypeStruct((NP, 128), jnp.float32),
        scratch_shapes=[pltpu.VMEM((NB, 1024), jnp.float32)],
        compiler_params=pltpu.CompilerParams(
            dimension_semantics=("parallel", "arbitrary", "arbitrary"),
            vmem_limit_bytes=56 * 1024 * 1024,
        ),
    )(x, w1, b1, w2, b2, wc, bc)


def kernel(P2, P3, P4, P5, rois, fc1_w, fc1_b, fc2_w, fc2_b, cls_w, cls_b,
           reg_w, reg_b):
    f = jnp.concatenate([
        P2[0].reshape(65536, 256), P3[0].reshape(16384, 256),
        P4[0].reshape(4096, 256), P5[0].reshape(1024, 256)], axis=0)
    f = jnp.pad(f, ((0, RPAD - 87040), (0, 0)))
    fa = f[:, :128].reshape(RPAD, 1, 128)
    fb = f[:, 128:].reshape(RPAD, 1, 128)

    r = jnp.pad(rois, ((0, NP - rois.shape[0]), (0, 0)))

    for i in range(2):
        pre = _prep(r)
        plo = _align(fa, pre, jnp.float32)
        phi = _align(fb, pre, jnp.float32)
        x = jnp.stack([plo, phi], axis=2).reshape(NP, 12544)
        wr = jnp.pad(reg_w[i], ((0, 0), (0, 124)))
        br = jnp.pad(reg_b[i], (0, 124)).reshape(1, 128)
        r = _head_delta(x, fc1_w[i], fc1_b[i].reshape(1, 1024), fc2_w[i],
                        fc2_b[i].reshape(1, 1024), wr, br, r)

    pre = _prep(r)
    plo = _align(fa, pre, jnp.bfloat16)
    phi = _align(fb, pre, jnp.bfloat16)
    x = jnp.stack([plo, phi], axis=2).reshape(NP, 12544)
    wc = jnp.pad(cls_w, ((0, 0), (0, 0), (0, 47)))
    bc = jnp.where(
        jnp.arange(128)[None, :] < 81,
        jnp.pad(cls_b, ((0, 0), (0, 47))), -1e9).reshape(3, 1, 128)
    probs = _head_ens(x, fc1_w, fc1_b.reshape(3, 1, 1024), fc2_w,
                      fc2_b.reshape(3, 1, 1024), wc, bc)
    return probs[:1000, :81]


# speedup vs baseline: 11.8530x; 1.0286x over previous
import jax
import jax.numpy as jnp
from jax.experimental import pallas as pl
from jax.experimental.pallas import tpu as pltpu

NP = 1024
RPAD = 87048
KB = 1792
KSTEPS = 7
NB = 256
IMG = 1024.0


def _prep(rois):
    y1, x1, y2, x2 = rois[:, 0], rois[:, 1], rois[:, 2], rois[:, 3]
    h = y2 - y1
    w = x2 - x1
    lvl = jnp.floor(4.0 + jnp.log2(jnp.sqrt(jnp.maximum(h * w, 1e-6)) / 224.0))
    li = jnp.clip(lvl, 2.0, 5.0).astype(jnp.int32) - 2
    stride = jnp.array([4.0, 8.0, 16.0, 32.0], jnp.float32)[li]
    hw = jnp.array([256, 128, 64, 32], jnp.int32)[li]
    off = jnp.array([0, 65536, 81920, 86016], jnp.int32)[li]
    s = rois / stride[:, None]
    grid = (jnp.arange(7, dtype=jnp.float32) + 0.5) / 7.0
    ys = s[:, 0:1] + grid[None, :] * (s[:, 2:3] - s[:, 0:1])
    xs = s[:, 1:2] + grid[None, :] * (s[:, 3:4] - s[:, 1:2])
    hwf = hw.astype(jnp.float32)[:, None]
    y0f = jnp.clip(jnp.floor(ys), 0.0, hwf - 1.0)
    x0f = jnp.clip(jnp.floor(xs), 0.0, hwf - 1.0)
    y0 = y0f.astype(jnp.int32)
    x0 = x0f.astype(jnp.int32)
    y1i = jnp.minimum(y0 + 1, hw[:, None] - 1)
    x1i = jnp.minimum(x0 + 1, hw[:, None] - 1)
    wy = jnp.clip(ys - y0f, 0.0, 1.0)
    wx = jnp.clip(xs - x0f, 0.0, 1.0)
    yb0 = off[:, None] + y0 * hw[:, None]
    yb1 = off[:, None] + y1i * hw[:, None]
    idx = jnp.concatenate([yb0, yb1, x0], axis=1)
    xe = x1i == x0
    cl = jnp.where(xe, 1.0, 1.0 - wx)
    ch = jnp.where(xe, 0.0, wx)
    a = (1.0 - wy)[:, :, None]
    b = wy[:, :, None]
    w0e = jnp.stack([a * cl[:, None, :], a * ch[:, None, :]], axis=-1)
    w1e = jnp.stack([b * cl[:, None, :], b * ch[:, None, :]], axis=-1)
    w0e = w0e.reshape(NP, 98, 1)
    w1e = w1e.reshape(NP, 98, 1)
    return idx, w0e, w1e


BA = 16


def _align_body(idx_sm, f_ref, w0_ref, w1_ref, out_ref, c0, c1):
    i = pl.program_id(0)
    for b in range(BA):
        row = i * BA + b
        xk = [idx_sm[row, 14 + k] for k in range(7)]
        for j in range(7):
            r0 = idx_sm[row, j]
            r1 = idx_sm[row, 7 + j]
            for k in range(7):
                p = b * 98 + 2 * (j * 7 + k)
                c0[p:p + 2, :] = f_ref[pl.ds(r0 + xk[k], 2), 0, :]
                c1[p:p + 2, :] = f_ref[pl.ds(r1 + xk[k], 2), 0, :]
    for b in range(BA):
        t = (c0[b * 98:(b + 1) * 98, :] * w0_ref[b]
             + c1[b * 98:(b + 1) * 98, :] * w1_ref[b])
        out_ref[b] = t.reshape(49, 2, 128).sum(axis=1).astype(out_ref.dtype)


def _align(f_half, pre, out_dtype):
    idx, w0e, w1e = pre
    grid_spec = pltpu.PrefetchScalarGridSpec(
        num_scalar_prefetch=1,
        grid=(NP // BA,),
        in_specs=[
            pl.BlockSpec(memory_space=pltpu.MemorySpace.VMEM),
            pl.BlockSpec((BA, 98, 1), lambda i, *_: (i, 0, 0)),
            pl.BlockSpec((BA, 98, 1), lambda i, *_: (i, 0, 0)),
        ],
        out_specs=pl.BlockSpec((BA, 49, 128), lambda i, *_: (i, 0, 0)),
        scratch_shapes=[pltpu.VMEM((BA * 98, 128), jnp.float32)] * 2,
    )
    return pl.pallas_call(
        _align_body,
        grid_spec=grid_spec,
        out_shape=jax.ShapeDtypeStruct((NP, 49, 128), out_dtype),
        compiler_params=pltpu.CompilerParams(
            dimension_semantics=("parallel",),
            vmem_limit_bytes=60 * 1024 * 1024,
        ),
    )(idx, f_half, w0e, w1e)


def _head_delta_body(x_ref, w1_ref, b1_ref, w2_ref, b2_ref, wr_ref, br_ref,
                     rois_ref, out_ref, acc):
    k = pl.program_id(1)

    @pl.when(k == 0)
    def _():
        acc[...] = jnp.zeros_like(acc)

    acc[...] += jnp.dot(x_ref[...].astype(jnp.bfloat16),
                        w1_ref[...].astype(jnp.bfloat16),
                        preferred_element_type=jnp.float32)

    @pl.when(k == KSTEPS - 1)
    def _():
        h1 = jnp.maximum(acc[...] + b1_ref[0], 0.0)
        h2 = jnp.maximum(
            jnp.dot(h1.astype(jnp.bfloat16), w2_ref[...].astype(jnp.bfloat16),
                    preferred_element_type=jnp.float32) + b2_ref[0], 0.0)
        dts = jnp.dot(h2.astype(jnp.bfloat16), wr_ref[...].astype(jnp.bfloat16),
                      preferred_element_type=jnp.float32) + br_ref[0]
        r = rois_ref[...]
        yy1, xx1, yy2, xx2 = r[:, 0:1], r[:, 1:2], r[:, 2:3], r[:, 3:4]
        hh = yy2 - yy1
        ww = xx2 - xx1
        cy = yy1 + 0.5 * hh + (dts[:, 0:1] * 0.1) * hh
        cx = xx1 + 0.5 * ww + (dts[:, 1:2] * 0.1) * ww
        nh = hh * jnp.exp(dts[:, 2:3] * 0.2)
        nw = ww * jnp.exp(dts[:, 3:4] * 0.2)
        nb = jnp.concatenate(
            [cy - 0.5 * nh, cx - 0.5 * nw, cy + 0.5 * nh, cx + 0.5 * nw], axis=1)
        out_ref[...] = jnp.clip(nb, 0.0, IMG)


def _head_delta(x, w1, b1, w2, b2, wr, br, rois):
    return pl.pallas_call(
        _head_delta_body,
        grid=(NP // NB, KSTEPS),
        in_specs=[
            pl.BlockSpec((NB, KB), lambda n, k: (n, k)),
            pl.BlockSpec((KB, 1024), lambda n, k: (k, 0)),
            pl.BlockSpec((1, 1024), lambda n, k: (0, 0)),
            pl.BlockSpec((1024, 1024), lambda n, k: (0, 0)),
            pl.BlockSpec((1, 1024), lambda n, k: (0, 0)),
            pl.BlockSpec((1024, 128), lambda n, k: (0, 0)),
            pl.BlockSpec((1, 128), lambda n, k: (0, 0)),
            pl.BlockSpec((NB, 4), lambda n, k: (n, 0)),
        ],
        out_specs=pl.BlockSpec((NB, 4), lambda n, k: (n, 0)),
        out_shape=jax.ShapeDtypeStruct((NP, 4), jnp.float32),
        scratch_shapes=[pltpu.VMEM((NB, 1024), jnp.float32)],
        compiler_params=pltpu.CompilerParams(
            dimension_semantics=("parallel", "arbitrary"),
            vmem_limit_bytes=56 * 1024 * 1024,
        ),
    )(x, w1, b1, w2, b2, wr, br, rois)


def _head_ens_body(x_ref, w1_ref, b1_ref, w2_ref, b2_ref, wc_ref, bc_ref,
                   out_ref, acc):
    h = pl.program_id(1)
    k = pl.program_id(2)

    @pl.when(k == 0)
    def _():
        acc[...] = jnp.zeros_like(acc)

    acc[...] += jnp.dot(x_ref[...],
                        w1_ref[0].astype(jnp.bfloat16),
                        preferred_element_type=jnp.float32)

    @pl.when(k == KSTEPS - 1)
    def _():
        h1 = jnp.maximum(acc[...] + b1_ref[0, 0], 0.0)
        h2 = jnp.maximum(
            jnp.dot(h1.astype(jnp.bfloat16), w2_ref[0].astype(jnp.bfloat16),
                    preferred_element_type=jnp.float32) + b2_ref[0, 0], 0.0)
        lg = jnp.dot(h2.astype(jnp.bfloat16), wc_ref[0].astype(jnp.bfloat16),
                     preferred_element_type=jnp.float32) + bc_ref[0, 0]
        m = jnp.max(lg, axis=-1, keepdims=True)
        e = jnp.exp(lg - m)
        p = e / (3.0 * jnp.sum(e, axis=-1, keepdims=True))

        @pl.when(h == 0)
        def _():
            out_ref[...] = p

        @pl.when(h != 0)
        def _():
            out_ref[...] = out_ref[...] + p


def _head_ens(x, w1, b1, w2, b2, wc, bc):
    return pl.pallas_call(
        _head_ens_body,
        grid=(NP // NB, 3, KSTEPS),
        in_specs=[
            pl.BlockSpec((NB, KB), lambda n, h, k: (n, k)),
            pl.BlockSpec((1, KB, 1024), lambda n, h, k: (h, k, 0)),
            pl.BlockSpec((1, 1, 1024), lambda n, h, k: (h, 0, 0)),
            pl.BlockSpec((1, 1024, 1024), lambda n, h, k: (h, 0, 0)),
            pl.BlockSpec((1, 1, 1024), lambda n, h, k: (h, 0, 0)),
            pl.BlockSpec((1, 1024, 128), lambda n, h, k: (h, 0, 0)),
            pl.BlockSpec((1, 1, 128), lambda n, h, k: (h, 0, 0)),
        ],
        out_specs=pl.BlockSpec((NB, 128), lambda n, h, k: (n, 0)),
        out_shape=jax.ShapeDtypeStruct((NP, 128), jnp.float32),
        scratch_shapes=[pltpu.VMEM((NB, 1024), jnp.float32)],
        compiler_params=pltpu.CompilerParams(
            dimension_semantics=("parallel", "arbitrary", "arbitrary"),
            vmem_limit_bytes=56 * 1024 * 1024,
        ),
    )(x, w1, b1, w2, b2, wc, bc)


def kernel(P2, P3, P4, P5, rois, fc1_w, fc1_b, fc2_w, fc2_b, cls_w, cls_b,
           reg_w, reg_b):
    f = jnp.concatenate([
        P2[0].reshape(65536, 256), P3[0].reshape(16384, 256),
        P4[0].reshape(4096, 256), P5[0].reshape(1024, 256)], axis=0)
    f = jnp.pad(f, ((0, RPAD - 87040), (0, 0)))
    fa = f[:, :128].reshape(RPAD, 1, 128)
    fb = f[:, 128:].reshape(RPAD, 1, 128)

    r = jnp.pad(rois, ((0, NP - rois.shape[0]), (0, 0)))

    for i in range(2):
        pre = _prep(r)
        plo = _align(fa, pre, jnp.float32)
        phi = _align(fb, pre, jnp.float32)
        x = jnp.stack([plo, phi], axis=2).reshape(NP, 12544)
        wr = jnp.pad(reg_w[i], ((0, 0), (0, 124)))
        br = jnp.pad(reg_b[i], (0, 124)).reshape(1, 128)
        r = _head_delta(x, fc1_w[i], fc1_b[i].reshape(1, 1024), fc2_w[i],
                        fc2_b[i].reshape(1, 1024), wr, br, r)

    pre = _prep(r)
    plo = _align(fa, pre, jnp.bfloat16)
    phi = _align(fb, pre, jnp.bfloat16)
    x = jnp.stack([plo, phi], axis=2).reshape(NP, 12544)
    wc = jnp.pad(cls_w, ((0, 0), (0, 0), (0, 47)))
    bc = jnp.where(
        jnp.arange(128)[None, :] < 81,
        jnp.pad(cls_b, ((0, 0), (0, 47))), -1e9).reshape(3, 1, 128)
    probs = _head_ens(x, fc1_w, fc1_b.reshape(3, 1, 1024), fc2_w,
                      fc2_b.reshape(3, 1, 1024), wc, bc)
    return probs[:1000, :81]
